# initial kernel scaffold (unmeasured)
import jax
import jax.numpy as jnp


def kernel(x, w_mat, scale_x, scale_w):
    print("PROBE x:", x.shape, x.dtype)
    print("PROBE w_mat:", w_mat.shape, w_mat.dtype)
    print("PROBE scale_x:", scale_x.shape, scale_x.dtype)
    print("PROBE scale_w:", scale_w.shape, scale_w.dtype)
    return jnp.zeros((512, 8192), jnp.float32)


# baseline (device time: 84551 ns/iter reference)
import jax
import jax.numpy as jnp
from jax import lax
from jax.experimental import pallas as pl
from jax.experimental.pallas import tpu as pltpu

N_DEV = 8
M_BLK = 512
K_BLK = 512
N_TOT = 8192
N_HALF = 4096
FP8 = jnp.float8_e4m3fn


def kernel(x, w_mat, scale_x, scale_w):
    def body(x_ref, w_hbm, sx_ref, sw_ref, out_hbm,
             x8, gather, wbuf, acc,
             send_sems, recv_sems, w_sems, out_sems):
        my = lax.axis_index("i")

        def w_dma(dstep, nh, slot):
            k = lax.rem(my + dstep, N_DEV)
            return pltpu.make_async_copy(
                w_hbm.at[pl.ds(k * K_BLK, K_BLK),
                         pl.ds(nh * N_HALF, N_HALF)],
                wbuf.at[slot],
                w_sems.at[slot],
            )

        pending = {}
        d0 = w_dma(0, 0, 0)
        d0.start()
        pending[0] = d0
        d1 = w_dma(1, 0, 1)
        d1.start()
        pending[1] = d1

        for j in range(N_DEV):
            x8[j] = x_ref[j * M_BLK:(j + 1) * M_BLK, :].astype(FP8)

        bar = pltpu.get_barrier_semaphore()
        for d in range(1, N_DEV):
            peer = lax.rem(my + d, N_DEV)
            pl.semaphore_signal(bar, inc=1, device_id=(peer,),
                                device_id_type=pl.DeviceIdType.MESH)
        pl.semaphore_wait(bar, N_DEV - 1)

        rdmas = []
        for d in range(1, N_DEV):
            dst = lax.rem(my + d, N_DEV)
            r = pltpu.make_async_remote_copy(
                src_ref=x8.at[dst],
                dst_ref=gather.at[my],
                send_sem=send_sems.at[d - 1],
                recv_sem=recv_sems.at[my],
                device_id=(dst,),
                device_id_type=pl.DeviceIdType.MESH,
            )
            r.start()
            rdmas.append(r)

        def wait_chunk(k):
            pltpu.make_async_remote_copy(
                src_ref=x8.at[k],
                dst_ref=gather.at[k],
                send_sem=send_sems.at[0],
                recv_sem=recv_sems.at[k],
                device_id=(my,),
                device_id_type=pl.DeviceIdType.MESH,
            ).wait_recv()

        s = sx_ref[0] * sw_ref[0]

        for nh in range(2):
            for dstep in range(N_DEV):
                t = nh * N_DEV + dstep
                k = lax.rem(my + dstep, N_DEV)
                if nh == 0 and dstep > 0:
                    wait_chunk(k)
                slot = t % 2
                pending[slot].wait()
                a = x8[my] if dstep == 0 else gather[k]
                b = wbuf[slot].astype(FP8)
                prod = lax.dot_general(
                    a, b, (((1,), (0,)), ((), ())),
                    preferred_element_type=jnp.float32,
                )
                if dstep == 0:
                    acc[nh] = prod
                else:
                    acc[nh] += prod
                if t + 2 < 2 * N_DEV:
                    nt = t + 2
                    nd = w_dma(nt % N_DEV, nt // N_DEV, slot)
                    nd.start()
                    pending[slot] = nd

            for c in range(4):
                sl = pl.ds(c * 1024, 1024)
                y = acc[nh, :, sl] * s
                acc[nh, :, sl] = y / (1.0 + jnp.exp(-jnp.clip(y, -60.0, 60.0)))

            out_dma = pltpu.make_async_copy(
                acc.at[nh],
                out_hbm.at[:, pl.ds(nh * N_HALF, N_HALF)],
                out_sems.at[nh],
            )
            out_dma.start()

        for nh in range(2):
            pltpu.make_async_copy(
                acc.at[nh],
                out_hbm.at[:, pl.ds(nh * N_HALF, N_HALF)],
                out_sems.at[nh],
            ).wait()
        for r in rdmas:
            r.wait_send()

    return pl.pallas_call(
        body,
        out_shape=jax.ShapeDtypeStruct((M_BLK, N_TOT), jnp.float32),
        in_specs=[
            pl.BlockSpec(memory_space=pltpu.VMEM),
            pl.BlockSpec(memory_space=pl.ANY),
            pl.BlockSpec(memory_space=pltpu.SMEM),
            pl.BlockSpec(memory_space=pltpu.SMEM),
        ],
        out_specs=pl.BlockSpec(memory_space=pl.ANY),
        scratch_shapes=[
            pltpu.VMEM((N_DEV, M_BLK, K_BLK), FP8),
            pltpu.VMEM((N_DEV, M_BLK, K_BLK), FP8),
            pltpu.VMEM((2, K_BLK, N_HALF), jnp.float32),
            pltpu.VMEM((2, M_BLK, N_HALF), jnp.float32),
            pltpu.SemaphoreType.DMA((N_DEV - 1,)),
            pltpu.SemaphoreType.DMA((N_DEV,)),
            pltpu.SemaphoreType.DMA((2,)),
            pltpu.SemaphoreType.DMA((2,)),
        ],
        compiler_params=pltpu.CompilerParams(
            collective_id=0, vmem_limit_bytes=100 * 1024 * 1024
        ),
    )(x, w_mat, scale_x, scale_w)


# device time: 76619 ns/iter; 1.1035x vs baseline; 1.1035x over previous
import jax
import jax.numpy as jnp
from jax import lax
from jax.experimental import pallas as pl
from jax.experimental.pallas import tpu as pltpu

N_DEV = 8
M_BLK = 512
K_BLK = 512
K_TOT = 4096
N_TOT = 8192
NB = 512
N_STRIPES = N_TOT // NB
S = 4
FP8 = jnp.float8_e4m3fn


def kernel(x, w_mat, scale_x, scale_w):
    def body(x_ref, w_hbm, sx_ref, sw_ref, out_hbm,
             xq, a8, wbuf, obuf,
             send_sems, recv_sems, w_sems, out_sems):
        my = lax.axis_index("i")

        def w_dma(t, slot):
            return pltpu.make_async_copy(
                w_hbm.at[:, pl.ds(t * NB, NB)],
                wbuf.at[slot],
                w_sems.at[slot],
            )

        pending = {}
        for t in range(S):
            d = w_dma(t, t)
            d.start()
            pending[t] = d

        xq[...] = x_ref[...].astype(FP8)
        a8[:, pl.ds(my * K_BLK, K_BLK)] = xq[pl.ds(my * M_BLK, M_BLK), :]

        bar = pltpu.get_barrier_semaphore()
        for d in range(1, N_DEV):
            peer = lax.rem(my + d, N_DEV)
            pl.semaphore_signal(bar, inc=1, device_id=(peer,),
                                device_id_type=pl.DeviceIdType.MESH)
        pl.semaphore_wait(bar, N_DEV - 1)

        rdmas = []
        for d in range(1, N_DEV):
            dst = lax.rem(my + d, N_DEV)
            r = pltpu.make_async_remote_copy(
                src_ref=xq.at[pl.ds(dst * M_BLK, M_BLK), :],
                dst_ref=a8.at[:, pl.ds(my * K_BLK, K_BLK)],
                send_sem=send_sems.at[d - 1],
                recv_sem=recv_sems.at[my],
                device_id=(dst,),
                device_id_type=pl.DeviceIdType.MESH,
            )
            r.start()
            rdmas.append(r)

        for d in range(1, N_DEV):
            k = lax.rem(my + d, N_DEV)
            pltpu.make_async_remote_copy(
                src_ref=xq.at[pl.ds(k * M_BLK, M_BLK), :],
                dst_ref=a8.at[:, pl.ds(k * K_BLK, K_BLK)],
                send_sem=send_sems.at[0],
                recv_sem=recv_sems.at[k],
                device_id=(my,),
                device_id_type=pl.DeviceIdType.MESH,
            ).wait_recv()

        s = sx_ref[0] * sw_ref[0]

        for t in range(N_STRIPES):
            slot = t % S
            oslot = t % 2
            pending[slot].wait()
            b = wbuf[slot].astype(FP8)
            prod = lax.dot_general(
                a8[...], b, (((1,), (0,)), ((), ())),
                preferred_element_type=jnp.float32,
            )
            if t + S < N_STRIPES:
                nd = w_dma(t + S, slot)
                nd.start()
                pending[slot] = nd
            y = prod * s
            res = y / (1.0 + jnp.exp(-jnp.clip(y, -60.0, 60.0)))
            if t >= 2:
                pltpu.make_async_copy(
                    obuf.at[oslot],
                    out_hbm.at[:, pl.ds((t - 2) * NB, NB)],
                    out_sems.at[oslot],
                ).wait()
            obuf[oslot] = res
            pltpu.make_async_copy(
                obuf.at[oslot],
                out_hbm.at[:, pl.ds(t * NB, NB)],
                out_sems.at[oslot],
            ).start()

        for t in (N_STRIPES - 2, N_STRIPES - 1):
            pltpu.make_async_copy(
                obuf.at[t % 2],
                out_hbm.at[:, pl.ds(t * NB, NB)],
                out_sems.at[t % 2],
            ).wait()
        for r in rdmas:
            r.wait_send()

    return pl.pallas_call(
        body,
        out_shape=jax.ShapeDtypeStruct((M_BLK, N_TOT), jnp.float32),
        in_specs=[
            pl.BlockSpec(memory_space=pltpu.VMEM),
            pl.BlockSpec(memory_space=pl.ANY),
            pl.BlockSpec(memory_space=pltpu.SMEM),
            pl.BlockSpec(memory_space=pltpu.SMEM),
        ],
        out_specs=pl.BlockSpec(memory_space=pl.ANY),
        scratch_shapes=[
            pltpu.VMEM((K_TOT, K_BLK), FP8),
            pltpu.VMEM((M_BLK, K_TOT), FP8),
            pltpu.VMEM((S, K_TOT, NB), jnp.float32),
            pltpu.VMEM((2, M_BLK, NB), jnp.float32),
            pltpu.SemaphoreType.DMA((N_DEV - 1,)),
            pltpu.SemaphoreType.DMA((N_DEV,)),
            pltpu.SemaphoreType.DMA((S,)),
            pltpu.SemaphoreType.DMA((2,)),
        ],
        compiler_params=pltpu.CompilerParams(
            collective_id=0, vmem_limit_bytes=100 * 1024 * 1024
        ),
    )(x, w_mat, scale_x, scale_w)
